# baseline (speedup 1.0000x reference)
import functools

import jax
import jax.numpy as jnp
from jax import lax
from jax.experimental import pallas as pl
from jax.experimental.pallas import tpu as pltpu
from jax.experimental.pallas import tpu_sc as plsc

N = 10000
E = 320000
F = 128
NC = 2
NS = 16
NW = NC * NS
EPW = E // NW
CH = 80
NCHUNK = EPW // CH
DPW = 640
NPAD = DPW * NS
RPW = N // NS
FH = F // NC
EPW2 = E // NS
NCHUNK2 = EPW2 // CH

_mesh = plsc.VectorSubcoreMesh(core_axis_name="c", subcore_axis_name="s")


def _rsqrt16(d):
    i = lax.bitcast_convert_type(d, jnp.int32)
    i = jnp.int32(0x5F3759DF) - lax.shift_right_logical(i, 1)
    y = lax.bitcast_convert_type(i, jnp.float32)
    for _ in range(3):
        y = y * (1.5 - 0.5 * d * y * y)
    return jnp.where(d > 0.0, y, 0.0)


@functools.partial(
    pl.kernel,
    out_type=jax.ShapeDtypeStruct((NW, NCHUNK, CH), jnp.float32),
    mesh=_mesh,
    compiler_params=pltpu.CompilerParams(
        needs_layout_passes=False, use_tc_tiling_on_sc=False),
    scratch_types=[
        pltpu.VMEM((NCHUNK, CH), jnp.int32),
        pltpu.VMEM((NCHUNK, CH), jnp.int32),
        pltpu.VMEM((NCHUNK, CH), jnp.float32),
        pltpu.VMEM((NCHUNK, CH), jnp.float32),
        pltpu.VMEM((NPAD,), jnp.float32),
        pltpu.VMEM((DPW,), jnp.float32),
        pltpu.VMEM_SHARED((NPAD,), jnp.float32),
        pltpu.VMEM_SHARED((NPAD,), jnp.float32),
    ],
)
def _wnorm_kernel(row_hbm, col_hbm, ew_hbm, wn_hbm,
                  rowv, colv, ewv, wnv, disv, dbuf, deg_sh, dis_sh):
    cid = lax.axis_index("c")
    sid = lax.axis_index("s")

    def _z(i, _):
        dbuf[pl.ds(i * 16, 16)] = jnp.zeros((16,), jnp.float32)
        return 0
    lax.fori_loop(0, DPW // 16, _z, 0)
    pltpu.sync_copy(dbuf, deg_sh.at[pl.ds(sid * DPW, DPW)])
    plsc.subcore_barrier()

    def _deg_block(w2):
        pltpu.sync_copy(row_hbm.at[w2], rowv)
        pltpu.sync_copy(ew_hbm.at[w2], ewv)

        def _sc(j, _):
            pltpu.sync_copy(ewv.at[j], deg_sh.at[rowv.at[j]], add=True)
            return 0
        lax.fori_loop(0, NCHUNK, _sc, 0)

    _deg_block(2 * sid)
    _deg_block(2 * sid + 1)
    plsc.subcore_barrier()

    pltpu.sync_copy(deg_sh.at[pl.ds(sid * DPW, DPW)], dbuf)

    def _rs(i, _):
        dbuf[pl.ds(i * 16, 16)] = _rsqrt16(dbuf[pl.ds(i * 16, 16)])
        return 0
    lax.fori_loop(0, DPW // 16, _rs, 0)
    pltpu.sync_copy(dbuf, dis_sh.at[pl.ds(sid * DPW, DPW)])
    plsc.subcore_barrier()

    wid = 2 * sid + cid
    pltpu.sync_copy(dis_sh, disv)
    pltpu.sync_copy(row_hbm.at[wid], rowv)
    pltpu.sync_copy(col_hbm.at[wid], colv)
    pltpu.sync_copy(ew_hbm.at[wid], ewv)

    def _wn(j, _):
        for c5 in range(CH // 16):
            sl = pl.ds(c5 * 16, 16)
            dr = plsc.load_gather(disv, [rowv[j, sl]])
            dc = plsc.load_gather(disv, [colv[j, sl]])
            wnv[j, sl] = -(dr * ewv[j, sl] * dc)
        return 0
    lax.fori_loop(0, NCHUNK, _wn, 0)
    pltpu.sync_copy(wnv, wn_hbm.at[wid])


@functools.partial(
    pl.kernel,
    out_type=jax.ShapeDtypeStruct((NC, N, FH), jnp.float32),
    mesh=_mesh,
    compiler_params=pltpu.CompilerParams(
        needs_layout_passes=False, use_tc_tiling_on_sc=False),
    scratch_types=[
        pltpu.VMEM((NCHUNK2, CH), jnp.int32),
        pltpu.VMEM((NCHUNK2, CH), jnp.int32),
        pltpu.VMEM((NCHUNK2, CH), jnp.float32),
        pltpu.VMEM((5, CH, FH), jnp.float32),
        pltpu.VMEM_SHARED((N, FH), jnp.float32),
        pltpu.SemaphoreType.DMA((5,)),
        pltpu.SemaphoreType.DMA((5,)),
    ],
)
def _spmm_kernel(z_hbm, row_hbm, col_hbm, w_hbm, zero_hbm, out_hbm,
                 rowv, colv, wv, rbuf, acc, gsem, ssem):
    cid = lax.axis_index("c")
    sid = lax.axis_index("s")

    pltpu.sync_copy(row_hbm.at[sid], rowv)
    pltpu.sync_copy(col_hbm.at[sid], colv)
    pltpu.sync_copy(w_hbm.at[sid], wv)
    pltpu.sync_copy(zero_hbm, acc.at[pl.ds(sid * RPW, RPW)])
    plsc.subcore_barrier()

    zh = z_hbm.at[cid]
    for p in range(4):
        pltpu.async_copy(zh.at[rowv.at[p]], rbuf.at[p], gsem.at[p])

    def _body(j, _):
        b = lax.rem(j, 5)

        pltpu.make_async_copy(zh.at[rowv.at[j]], rbuf.at[b],
                              gsem.at[b]).wait()

        for g in range(CH // 16):
            wvec = wv[j, pl.ds(g * 16, 16)]
            for t in range(16):
                ws = wvec[t]
                cc = g * 16 + t
                for k in range(FH // 16):
                    sl = pl.ds(k * 16, 16)
                    rbuf[b, cc, sl] = rbuf[b, cc, sl] * ws

        @pl.when(j >= 1)
        def _():
            bp = lax.rem(j - 1, 5)
            pltpu.make_async_copy(rbuf.at[bp], acc.at[colv.at[j - 1]],
                                  ssem.at[bp]).wait()

        @pl.when(j + 4 < NCHUNK2)
        def _():
            bg = lax.rem(j + 4, 5)
            pltpu.async_copy(zh.at[rowv.at[j + 4]], rbuf.at[bg],
                             gsem.at[bg])

        pltpu.async_copy(rbuf.at[b], acc.at[colv.at[j]], ssem.at[b],
                         add=True)
        return 0
    lax.fori_loop(0, NCHUNK2, _body, 0)

    t = NCHUNK2 - 1
    pltpu.make_async_copy(rbuf.at[t % 5], acc.at[colv.at[t]],
                          ssem.at[t % 5]).wait()
    plsc.subcore_barrier()

    pltpu.sync_copy(acc.at[pl.ds(sid * RPW, RPW)],
                    out_hbm.at[cid, pl.ds(sid * RPW, RPW)])


def _fin_body(x_ref, t1_ref, u_ref, w_ref, b_ref, o_ref):
    t1 = jnp.concatenate([t1_ref[0], t1_ref[1]], axis=-1)
    u = jnp.concatenate([u_ref[0], u_ref[1]], axis=-1)
    acc = jnp.dot(x_ref[...], w_ref[0] - w_ref[2],
                  preferred_element_type=jnp.float32)
    acc += jnp.dot(t1, w_ref[1], preferred_element_type=jnp.float32)
    acc += jnp.dot(u, 2.0 * w_ref[2], preferred_element_type=jnp.float32)
    o_ref[...] = acc + b_ref[...]


def kernel(x, edge_index, edge_weight, W, bias):
    row3 = edge_index[0].reshape(NW, NCHUNK, CH)
    col3 = edge_index[1].reshape(NW, NCHUNK, CH)
    ew3 = edge_weight.reshape(NW, NCHUNK, CH)
    rowS = edge_index[0].reshape(NS, NCHUNK2, CH)
    colS = edge_index[1].reshape(NS, NCHUNK2, CH)
    zeros = jnp.zeros((RPW, FH), jnp.float32)

    wn3 = _wnorm_kernel(row3, col3, ew3)
    wnS = wn3.reshape(NS, NCHUNK2, CH)
    xh = jnp.stack([x[:, :FH], x[:, FH:]])
    P = _spmm_kernel(xh, rowS, colS, wnS, zeros)
    U = _spmm_kernel(P, rowS, colS, wnS, zeros)

    bn = 1000
    out = pl.pallas_call(
        _fin_body,
        grid=(N // bn,),
        in_specs=[
            pl.BlockSpec((bn, F), lambda i: (i, 0)),
            pl.BlockSpec((NC, bn, FH), lambda i: (0, i, 0)),
            pl.BlockSpec((NC, bn, FH), lambda i: (0, i, 0)),
            pl.BlockSpec((3, F, F), lambda i: (0, 0, 0)),
            pl.BlockSpec((1, F), lambda i: (0, 0)),
        ],
        out_specs=pl.BlockSpec((bn, F), lambda i: (i, 0)),
        out_shape=jax.ShapeDtypeStruct((N, F), jnp.float32),
    )(x, P, U, W, bias[None, :])
    return out


# speedup vs baseline: 1.0010x; 1.0010x over previous
import functools

import jax
import jax.numpy as jnp
from jax import lax
from jax.experimental import pallas as pl
from jax.experimental.pallas import tpu as pltpu
from jax.experimental.pallas import tpu_sc as plsc

N = 10000
E = 320000
F = 128
NC = 2
NS = 16
NW = NC * NS
EPW = E // NW
CH = 80
NCHUNK = EPW // CH
DPW = 640
NPAD = DPW * NS
RPW = N // NS
FH = F // NC
EPW2 = E // NS
NCHUNK2 = EPW2 // CH

_mesh = plsc.VectorSubcoreMesh(core_axis_name="c", subcore_axis_name="s")


def _rsqrt16(d):
    i = lax.bitcast_convert_type(d, jnp.int32)
    i = jnp.int32(0x5F3759DF) - lax.shift_right_logical(i, 1)
    y = lax.bitcast_convert_type(i, jnp.float32)
    for _ in range(3):
        y = y * (1.5 - 0.5 * d * y * y)
    return jnp.where(d > 0.0, y, 0.0)


@functools.partial(
    pl.kernel,
    out_type=jax.ShapeDtypeStruct((NW, NCHUNK, CH), jnp.float32),
    mesh=_mesh,
    compiler_params=pltpu.CompilerParams(
        needs_layout_passes=False, use_tc_tiling_on_sc=False),
    scratch_types=[
        pltpu.VMEM((NCHUNK, CH), jnp.int32),
        pltpu.VMEM((NCHUNK, CH), jnp.int32),
        pltpu.VMEM((NCHUNK, CH), jnp.float32),
        pltpu.VMEM((NCHUNK, CH), jnp.float32),
        pltpu.VMEM((NPAD,), jnp.float32),
        pltpu.VMEM((DPW,), jnp.float32),
        pltpu.VMEM_SHARED((NPAD,), jnp.float32),
        pltpu.VMEM_SHARED((NPAD,), jnp.float32),
    ],
)
def _wnorm_kernel(row_hbm, col_hbm, ew_hbm, wn_hbm,
                  rowv, colv, ewv, wnv, disv, dbuf, deg_sh, dis_sh):
    cid = lax.axis_index("c")
    sid = lax.axis_index("s")

    def _z(i, _):
        dbuf[pl.ds(i * 16, 16)] = jnp.zeros((16,), jnp.float32)
        return 0
    lax.fori_loop(0, DPW // 16, _z, 0)
    pltpu.sync_copy(dbuf, deg_sh.at[pl.ds(sid * DPW, DPW)])
    plsc.subcore_barrier()

    def _deg_block(w2):
        pltpu.sync_copy(row_hbm.at[w2], rowv)
        pltpu.sync_copy(ew_hbm.at[w2], ewv)

        def _sc(j, _):
            pltpu.sync_copy(ewv.at[j], deg_sh.at[rowv.at[j]], add=True)
            return 0
        lax.fori_loop(0, NCHUNK, _sc, 0)

    _deg_block(2 * sid)
    _deg_block(2 * sid + 1)
    plsc.subcore_barrier()

    pltpu.sync_copy(deg_sh.at[pl.ds(sid * DPW, DPW)], dbuf)

    def _rs(i, _):
        dbuf[pl.ds(i * 16, 16)] = _rsqrt16(dbuf[pl.ds(i * 16, 16)])
        return 0
    lax.fori_loop(0, DPW // 16, _rs, 0)
    pltpu.sync_copy(dbuf, dis_sh.at[pl.ds(sid * DPW, DPW)])
    plsc.subcore_barrier()

    wid = 2 * sid + cid
    pltpu.sync_copy(dis_sh, disv)
    pltpu.sync_copy(row_hbm.at[wid], rowv)
    pltpu.sync_copy(col_hbm.at[wid], colv)
    pltpu.sync_copy(ew_hbm.at[wid], ewv)

    def _wn(j, _):
        for c5 in range(CH // 16):
            sl = pl.ds(c5 * 16, 16)
            dr = plsc.load_gather(disv, [rowv[j, sl]])
            dc = plsc.load_gather(disv, [colv[j, sl]])
            wnv[j, sl] = -(dr * ewv[j, sl] * dc)
        return 0
    lax.fori_loop(0, NCHUNK, _wn, 0)
    pltpu.sync_copy(wnv, wn_hbm.at[wid])


@functools.partial(
    pl.kernel,
    out_type=jax.ShapeDtypeStruct((NC, N, FH), jnp.float32),
    mesh=_mesh,
    compiler_params=pltpu.CompilerParams(
        needs_layout_passes=False, use_tc_tiling_on_sc=False),
    scratch_types=[
        pltpu.VMEM((NCHUNK2, CH), jnp.int32),
        pltpu.VMEM((NCHUNK2, CH), jnp.int32),
        pltpu.VMEM((NCHUNK2, CH), jnp.float32),
        pltpu.VMEM((4, CH, FH), jnp.float32),
        pltpu.VMEM_SHARED((N, FH), jnp.float32),
        pltpu.SemaphoreType.DMA((4,)),
        pltpu.SemaphoreType.DMA((4,)),
    ],
)
def _spmm_kernel(z_hbm, row_hbm, col_hbm, w_hbm, zero_hbm, out_hbm,
                 rowv, colv, wv, rbuf, acc, gsem, ssem):
    cid = lax.axis_index("c")
    sid = lax.axis_index("s")

    pltpu.sync_copy(row_hbm.at[sid], rowv)
    pltpu.sync_copy(col_hbm.at[sid], colv)
    pltpu.sync_copy(w_hbm.at[sid], wv)
    pltpu.sync_copy(zero_hbm, acc.at[pl.ds(sid * RPW, RPW)])
    plsc.subcore_barrier()

    zh = z_hbm.at[cid]
    for p in range(3):
        pltpu.async_copy(zh.at[rowv.at[p]], rbuf.at[p], gsem.at[p])

    def _body(j, _):
        b = lax.rem(j, 4)

        pltpu.make_async_copy(zh.at[rowv.at[j]], rbuf.at[b],
                              gsem.at[b]).wait()

        for g in range(CH // 16):
            wvec = wv[j, pl.ds(g * 16, 16)]
            for t in range(16):
                ws = wvec[t]
                cc = g * 16 + t
                for k in range(FH // 16):
                    sl = pl.ds(k * 16, 16)
                    rbuf[b, cc, sl] = rbuf[b, cc, sl] * ws

        @pl.when(j >= 1)
        def _():
            bp = lax.rem(j - 1, 4)
            pltpu.make_async_copy(rbuf.at[bp], acc.at[colv.at[j - 1]],
                                  ssem.at[bp]).wait()

        @pl.when(j + 3 < NCHUNK2)
        def _():
            bg = lax.rem(j + 3, 4)
            pltpu.async_copy(zh.at[rowv.at[j + 3]], rbuf.at[bg],
                             gsem.at[bg])

        pltpu.async_copy(rbuf.at[b], acc.at[colv.at[j]], ssem.at[b],
                         add=True)
        return 0
    lax.fori_loop(0, NCHUNK2, _body, 0)

    t = NCHUNK2 - 1
    pltpu.make_async_copy(rbuf.at[t % 4], acc.at[colv.at[t]],
                          ssem.at[t % 4]).wait()
    plsc.subcore_barrier()

    pltpu.sync_copy(acc.at[pl.ds(sid * RPW, RPW)],
                    out_hbm.at[cid, pl.ds(sid * RPW, RPW)])


def _fin_body(x_ref, t1_ref, u_ref, w_ref, b_ref, o_ref):
    t1 = jnp.concatenate([t1_ref[0], t1_ref[1]], axis=-1)
    u = jnp.concatenate([u_ref[0], u_ref[1]], axis=-1)
    acc = jnp.dot(x_ref[...], w_ref[0] - w_ref[2],
                  preferred_element_type=jnp.float32)
    acc += jnp.dot(t1, w_ref[1], preferred_element_type=jnp.float32)
    acc += jnp.dot(u, 2.0 * w_ref[2], preferred_element_type=jnp.float32)
    o_ref[...] = acc + b_ref[...]


def kernel(x, edge_index, edge_weight, W, bias):
    row3 = edge_index[0].reshape(NW, NCHUNK, CH)
    col3 = edge_index[1].reshape(NW, NCHUNK, CH)
    ew3 = edge_weight.reshape(NW, NCHUNK, CH)
    rowS = edge_index[0].reshape(NS, NCHUNK2, CH)
    colS = edge_index[1].reshape(NS, NCHUNK2, CH)
    zeros = jnp.zeros((RPW, FH), jnp.float32)

    wn3 = _wnorm_kernel(row3, col3, ew3)
    wnS = wn3.reshape(NS, NCHUNK2, CH)
    xh = jnp.stack([x[:, :FH], x[:, FH:]])
    P = _spmm_kernel(xh, rowS, colS, wnS, zeros)
    U = _spmm_kernel(P, rowS, colS, wnS, zeros)

    bn = 1000
    out = pl.pallas_call(
        _fin_body,
        grid=(N // bn,),
        in_specs=[
            pl.BlockSpec((bn, F), lambda i: (i, 0)),
            pl.BlockSpec((NC, bn, FH), lambda i: (0, i, 0)),
            pl.BlockSpec((NC, bn, FH), lambda i: (0, i, 0)),
            pl.BlockSpec((3, F, F), lambda i: (0, 0, 0)),
            pl.BlockSpec((1, F), lambda i: (0, 0)),
        ],
        out_specs=pl.BlockSpec((bn, F), lambda i: (i, 0)),
        out_shape=jax.ShapeDtypeStruct((N, F), jnp.float32),
    )(x, P, U, W, bias[None, :])
    return out


# speedup vs baseline: 1.0481x; 1.0470x over previous
import functools

import jax
import jax.numpy as jnp
from jax import lax
from jax.experimental import pallas as pl
from jax.experimental.pallas import tpu as pltpu
from jax.experimental.pallas import tpu_sc as plsc

N = 10000
E = 320000
F = 128
NC = 2
NS = 16
NW = NC * NS
EPW = E // NW
CH = 80
NCHUNK = EPW // CH
DPW = 640
NPAD = DPW * NS
RPW = N // NS
FH = F // NC
EPW2 = E // NS
NCHUNK2 = EPW2 // CH

_mesh = plsc.VectorSubcoreMesh(core_axis_name="c", subcore_axis_name="s")


def _rsqrt16(d):
    i = lax.bitcast_convert_type(d, jnp.int32)
    i = jnp.int32(0x5F3759DF) - lax.shift_right_logical(i, 1)
    y = lax.bitcast_convert_type(i, jnp.float32)
    for _ in range(3):
        y = y * (1.5 - 0.5 * d * y * y)
    return jnp.where(d > 0.0, y, 0.0)


@functools.partial(
    pl.kernel,
    out_type=jax.ShapeDtypeStruct((NW, NCHUNK, CH), jnp.float32),
    mesh=_mesh,
    compiler_params=pltpu.CompilerParams(
        needs_layout_passes=False, use_tc_tiling_on_sc=False),
    scratch_types=[
        pltpu.VMEM((NCHUNK, CH), jnp.int32),
        pltpu.VMEM((NCHUNK, CH), jnp.int32),
        pltpu.VMEM((NCHUNK, CH), jnp.float32),
        pltpu.VMEM((NCHUNK, CH), jnp.float32),
        pltpu.VMEM((NPAD,), jnp.float32),
        pltpu.VMEM((DPW,), jnp.float32),
        pltpu.VMEM_SHARED((NPAD,), jnp.float32),
        pltpu.VMEM_SHARED((NPAD,), jnp.float32),
        pltpu.SemaphoreType.DMA,
    ],
)
def _wnorm_kernel(row_hbm, col_hbm, ew_hbm, wn_hbm,
                  rowv, colv, ewv, wnv, disv, dbuf, deg_sh, dis_sh, dsem):
    cid = lax.axis_index("c")
    sid = lax.axis_index("s")

    def _z(i, _):
        dbuf[pl.ds(i * 16, 16)] = jnp.zeros((16,), jnp.float32)
        return 0
    lax.fori_loop(0, DPW // 16, _z, 0)
    pltpu.sync_copy(dbuf, deg_sh.at[pl.ds(sid * DPW, DPW)])
    plsc.subcore_barrier()

    def _deg_block(w2):
        pltpu.sync_copy(row_hbm.at[w2], rowv)
        pltpu.sync_copy(ew_hbm.at[w2], ewv)

        def _sc(bb, _):
            base = bb * 25
            for jj in range(25):
                pltpu.async_copy(ewv.at[base + jj],
                                 deg_sh.at[rowv.at[base + jj]], dsem,
                                 add=True)
            for jj in range(25):
                pltpu.make_async_copy(ewv.at[base + jj],
                                      deg_sh.at[rowv.at[base + jj]],
                                      dsem).wait()
            return 0
        lax.fori_loop(0, NCHUNK // 25, _sc, 0)

    _deg_block(2 * sid)
    _deg_block(2 * sid + 1)
    plsc.subcore_barrier()

    pltpu.sync_copy(deg_sh.at[pl.ds(sid * DPW, DPW)], dbuf)

    def _rs(i, _):
        dbuf[pl.ds(i * 16, 16)] = _rsqrt16(dbuf[pl.ds(i * 16, 16)])
        return 0
    lax.fori_loop(0, DPW // 16, _rs, 0)
    pltpu.sync_copy(dbuf, dis_sh.at[pl.ds(sid * DPW, DPW)])
    plsc.subcore_barrier()

    wid = 2 * sid + cid
    pltpu.sync_copy(dis_sh, disv)
    pltpu.sync_copy(row_hbm.at[wid], rowv)
    pltpu.sync_copy(col_hbm.at[wid], colv)
    pltpu.sync_copy(ew_hbm.at[wid], ewv)

    def _wn(j, _):
        for c5 in range(CH // 16):
            sl = pl.ds(c5 * 16, 16)
            dr = plsc.load_gather(disv, [rowv[j, sl]])
            dc = plsc.load_gather(disv, [colv[j, sl]])
            wnv[j, sl] = -(dr * ewv[j, sl] * dc)
        return 0
    lax.fori_loop(0, NCHUNK, _wn, 0)
    pltpu.sync_copy(wnv, wn_hbm.at[wid])


@functools.partial(
    pl.kernel,
    out_type=jax.ShapeDtypeStruct((NC, N, FH), jnp.float32),
    mesh=_mesh,
    compiler_params=pltpu.CompilerParams(
        needs_layout_passes=False, use_tc_tiling_on_sc=False),
    scratch_types=[
        pltpu.VMEM((NCHUNK2, CH), jnp.int32),
        pltpu.VMEM((NCHUNK2, CH), jnp.int32),
        pltpu.VMEM((NCHUNK2, CH), jnp.float32),
        pltpu.VMEM((4, CH, FH), jnp.float32),
        pltpu.VMEM_SHARED((N, FH), jnp.float32),
        pltpu.SemaphoreType.DMA((4,)),
        pltpu.SemaphoreType.DMA((4,)),
    ],
)
def _spmm_kernel(z_hbm, row_hbm, col_hbm, w_hbm, zero_hbm, out_hbm,
                 rowv, colv, wv, rbuf, acc, gsem, ssem):
    cid = lax.axis_index("c")
    sid = lax.axis_index("s")

    pltpu.sync_copy(row_hbm.at[sid], rowv)
    pltpu.sync_copy(col_hbm.at[sid], colv)
    pltpu.sync_copy(w_hbm.at[sid], wv)
    pltpu.sync_copy(zero_hbm, acc.at[pl.ds(sid * RPW, RPW)])
    plsc.subcore_barrier()

    zh = z_hbm.at[cid]
    for p in range(3):
        pltpu.async_copy(zh.at[rowv.at[p]], rbuf.at[p], gsem.at[p])

    def _body(j, _):
        b = lax.rem(j, 4)

        pltpu.make_async_copy(zh.at[rowv.at[j]], rbuf.at[b],
                              gsem.at[b]).wait()

        for g in range(CH // 16):
            wvec = wv[j, pl.ds(g * 16, 16)]
            for t in range(16):
                ws = wvec[t]
                cc = g * 16 + t
                for k in range(FH // 16):
                    sl = pl.ds(k * 16, 16)
                    rbuf[b, cc, sl] = rbuf[b, cc, sl] * ws

        @pl.when(j >= 1)
        def _():
            bp = lax.rem(j - 1, 4)
            pltpu.make_async_copy(rbuf.at[bp], acc.at[colv.at[j - 1]],
                                  ssem.at[bp]).wait()

        @pl.when(j + 3 < NCHUNK2)
        def _():
            bg = lax.rem(j + 3, 4)
            pltpu.async_copy(zh.at[rowv.at[j + 3]], rbuf.at[bg],
                             gsem.at[bg])

        pltpu.async_copy(rbuf.at[b], acc.at[colv.at[j]], ssem.at[b],
                         add=True)
        return 0
    lax.fori_loop(0, NCHUNK2, _body, 0)

    t = NCHUNK2 - 1
    pltpu.make_async_copy(rbuf.at[t % 4], acc.at[colv.at[t]],
                          ssem.at[t % 4]).wait()
    plsc.subcore_barrier()

    pltpu.sync_copy(acc.at[pl.ds(sid * RPW, RPW)],
                    out_hbm.at[cid, pl.ds(sid * RPW, RPW)])


def _fin_body(x_ref, t1_ref, u_ref, w_ref, b_ref, o_ref):
    t1 = jnp.concatenate([t1_ref[0], t1_ref[1]], axis=-1)
    u = jnp.concatenate([u_ref[0], u_ref[1]], axis=-1)
    acc = jnp.dot(x_ref[...], w_ref[0] - w_ref[2],
                  preferred_element_type=jnp.float32)
    acc += jnp.dot(t1, w_ref[1], preferred_element_type=jnp.float32)
    acc += jnp.dot(u, 2.0 * w_ref[2], preferred_element_type=jnp.float32)
    o_ref[...] = acc + b_ref[...]


def kernel(x, edge_index, edge_weight, W, bias):
    row3 = edge_index[0].reshape(NW, NCHUNK, CH)
    col3 = edge_index[1].reshape(NW, NCHUNK, CH)
    ew3 = edge_weight.reshape(NW, NCHUNK, CH)
    rowS = edge_index[0].reshape(NS, NCHUNK2, CH)
    colS = edge_index[1].reshape(NS, NCHUNK2, CH)
    zeros = jnp.zeros((RPW, FH), jnp.float32)

    wn3 = _wnorm_kernel(row3, col3, ew3)
    wnS = wn3.reshape(NS, NCHUNK2, CH)
    xh = jnp.stack([x[:, :FH], x[:, FH:]])
    P = _spmm_kernel(xh, rowS, colS, wnS, zeros)
    U = _spmm_kernel(P, rowS, colS, wnS, zeros)

    bn = 1000
    out = pl.pallas_call(
        _fin_body,
        grid=(N // bn,),
        in_specs=[
            pl.BlockSpec((bn, F), lambda i: (i, 0)),
            pl.BlockSpec((NC, bn, FH), lambda i: (0, i, 0)),
            pl.BlockSpec((NC, bn, FH), lambda i: (0, i, 0)),
            pl.BlockSpec((3, F, F), lambda i: (0, 0, 0)),
            pl.BlockSpec((1, F), lambda i: (0, 0)),
        ],
        out_specs=pl.BlockSpec((bn, F), lambda i: (i, 0)),
        out_shape=jax.ShapeDtypeStruct((N, F), jnp.float32),
    )(x, P, U, W, bias[None, :])
    return out


# speedup vs baseline: 1.0493x; 1.0011x over previous
import functools

import jax
import jax.numpy as jnp
from jax import lax
from jax.experimental import pallas as pl
from jax.experimental.pallas import tpu as pltpu
from jax.experimental.pallas import tpu_sc as plsc

N = 10000
E = 320000
F = 128
NC = 2
NS = 16
NW = NC * NS
EPW = E // NW
CH = 80
NCHUNK = EPW // CH
DPW = 640
NPAD = DPW * NS
RPW = N // NS
FH = F // NC
EPW2 = E // NS
NCHUNK2 = EPW2 // CH

_mesh = plsc.VectorSubcoreMesh(core_axis_name="c", subcore_axis_name="s")


def _rsqrt16(d):
    i = lax.bitcast_convert_type(d, jnp.int32)
    i = jnp.int32(0x5F3759DF) - lax.shift_right_logical(i, 1)
    y = lax.bitcast_convert_type(i, jnp.float32)
    for _ in range(3):
        y = y * (1.5 - 0.5 * d * y * y)
    return jnp.where(d > 0.0, y, 0.0)


@functools.partial(
    pl.kernel,
    out_type=jax.ShapeDtypeStruct((NW, NCHUNK, CH), jnp.float32),
    mesh=_mesh,
    compiler_params=pltpu.CompilerParams(
        needs_layout_passes=False, use_tc_tiling_on_sc=False),
    scratch_types=[
        pltpu.VMEM((NCHUNK, CH), jnp.int32),
        pltpu.VMEM((NCHUNK, CH), jnp.int32),
        pltpu.VMEM((NCHUNK, CH), jnp.float32),
        pltpu.VMEM((NCHUNK, CH), jnp.float32),
        pltpu.VMEM((NPAD,), jnp.float32),
        pltpu.VMEM((DPW,), jnp.float32),
        pltpu.VMEM_SHARED((NPAD,), jnp.float32),
        pltpu.VMEM_SHARED((NPAD,), jnp.float32),
        pltpu.SemaphoreType.DMA,
    ],
)
def _wnorm_kernel(row_hbm, col_hbm, ew_hbm, wn_hbm,
                  rowv, colv, ewv, wnv, disv, dbuf, deg_sh, dis_sh, dsem):
    cid = lax.axis_index("c")
    sid = lax.axis_index("s")

    def _z(i, _):
        dbuf[pl.ds(i * 16, 16)] = jnp.zeros((16,), jnp.float32)
        return 0
    lax.fori_loop(0, DPW // 16, _z, 0)
    pltpu.sync_copy(dbuf, deg_sh.at[pl.ds(sid * DPW, DPW)])
    plsc.subcore_barrier()

    def _deg_block(w2):
        pltpu.sync_copy(row_hbm.at[w2], rowv)
        pltpu.sync_copy(ew_hbm.at[w2], ewv)

        def _sc(bb, _):
            base = bb * 25
            for jj in range(25):
                pltpu.async_copy(ewv.at[base + jj],
                                 deg_sh.at[rowv.at[base + jj]], dsem,
                                 add=True)
            for jj in range(25):
                pltpu.make_async_copy(ewv.at[base + jj],
                                      deg_sh.at[rowv.at[base + jj]],
                                      dsem).wait()
            return 0
        lax.fori_loop(0, NCHUNK // 25, _sc, 0)

    _deg_block(2 * sid)
    _deg_block(2 * sid + 1)
    plsc.subcore_barrier()

    pltpu.sync_copy(deg_sh.at[pl.ds(sid * DPW, DPW)], dbuf)

    def _rs(i, _):
        dbuf[pl.ds(i * 16, 16)] = _rsqrt16(dbuf[pl.ds(i * 16, 16)])
        return 0
    lax.fori_loop(0, DPW // 16, _rs, 0)
    pltpu.sync_copy(dbuf, dis_sh.at[pl.ds(sid * DPW, DPW)])
    plsc.subcore_barrier()

    wid = 2 * sid + cid
    pltpu.sync_copy(dis_sh, disv)
    pltpu.sync_copy(row_hbm.at[wid], rowv)
    pltpu.sync_copy(col_hbm.at[wid], colv)
    pltpu.sync_copy(ew_hbm.at[wid], ewv)

    def _wn(j, _):
        for c5 in range(CH // 16):
            sl = pl.ds(c5 * 16, 16)
            dr = plsc.load_gather(disv, [rowv[j, sl]])
            dc = plsc.load_gather(disv, [colv[j, sl]])
            wnv[j, sl] = -(dr * ewv[j, sl] * dc)
        return 0
    lax.fori_loop(0, NCHUNK, _wn, 0)
    pltpu.sync_copy(wnv, wn_hbm.at[wid])


@functools.partial(
    pl.kernel,
    out_type=jax.ShapeDtypeStruct((NC, N, FH), jnp.float32),
    mesh=_mesh,
    compiler_params=pltpu.CompilerParams(
        needs_layout_passes=False, use_tc_tiling_on_sc=False),
    scratch_types=[
        pltpu.VMEM((NCHUNK2, CH), jnp.int32),
        pltpu.VMEM((NCHUNK2, CH), jnp.int32),
        pltpu.VMEM((NCHUNK2, CH), jnp.float32),
        pltpu.VMEM((4, CH, FH), jnp.float32),
        pltpu.VMEM_SHARED((N, FH), jnp.float32),
        pltpu.SemaphoreType.DMA((4,)),
        pltpu.SemaphoreType.DMA((4,)),
    ],
)
def _spmm_kernel(z_hbm, row_hbm, col_hbm, w_hbm, zero_hbm, out_hbm,
                 rowv, colv, wv, rbuf, acc, gsem, ssem):
    cid = lax.axis_index("c")
    sid = lax.axis_index("s")

    pltpu.sync_copy(row_hbm.at[sid], rowv)
    pltpu.sync_copy(col_hbm.at[sid], colv)
    pltpu.sync_copy(w_hbm.at[sid], wv)
    pltpu.sync_copy(zero_hbm, acc.at[pl.ds(sid * RPW, RPW)])
    plsc.subcore_barrier()

    zh = z_hbm.at[cid]
    for p in range(3):
        pltpu.async_copy(zh.at[rowv.at[p]], rbuf.at[p], gsem.at[p])

    def _body(j, _):
        b = lax.rem(j, 4)

        pltpu.make_async_copy(zh.at[rowv.at[j]], rbuf.at[b],
                              gsem.at[b]).wait()

        for g in range(CH // 16):
            wvec = wv[j, pl.ds(g * 16, 16)]
            for t in range(16):
                ws = wvec[t]
                cc = g * 16 + t
                for k in range(FH // 16):
                    sl = pl.ds(k * 16, 16)
                    rbuf[b, cc, sl] = rbuf[b, cc, sl] * ws

        @pl.when(j >= 1)
        def _():
            bp = lax.rem(j - 1, 4)
            pltpu.make_async_copy(rbuf.at[bp], acc.at[colv.at[j - 1]],
                                  ssem.at[bp]).wait()

        @pl.when(j + 3 < NCHUNK2)
        def _():
            bg = lax.rem(j + 3, 4)
            pltpu.async_copy(zh.at[rowv.at[j + 3]], rbuf.at[bg],
                             gsem.at[bg])

        pltpu.async_copy(rbuf.at[b], acc.at[colv.at[j]], ssem.at[b],
                         add=True)
        return 0
    lax.fori_loop(0, NCHUNK2, _body, 0)

    t = NCHUNK2 - 1
    pltpu.make_async_copy(rbuf.at[t % 4], acc.at[colv.at[t]],
                          ssem.at[t % 4]).wait()
    plsc.subcore_barrier()

    pltpu.sync_copy(acc.at[pl.ds(sid * RPW, RPW)],
                    out_hbm.at[cid, pl.ds(sid * RPW, RPW)])


def _part_body(x_ref, t1_ref, w_ref, b_ref, o_ref):
    t1 = jnp.concatenate([t1_ref[0], t1_ref[1]], axis=-1)
    acc = jnp.dot(x_ref[...], w_ref[0] - w_ref[2],
                  preferred_element_type=jnp.float32)
    acc += jnp.dot(t1, w_ref[1], preferred_element_type=jnp.float32)
    o_ref[...] = acc + b_ref[...]


def _fin_body(p_ref, u_ref, w_ref, o_ref):
    u = jnp.concatenate([u_ref[0], u_ref[1]], axis=-1)
    o_ref[...] = p_ref[...] + jnp.dot(u, 2.0 * w_ref[2],
                                      preferred_element_type=jnp.float32)


def kernel(x, edge_index, edge_weight, W, bias):
    row3 = edge_index[0].reshape(NW, NCHUNK, CH)
    col3 = edge_index[1].reshape(NW, NCHUNK, CH)
    ew3 = edge_weight.reshape(NW, NCHUNK, CH)
    rowS = edge_index[0].reshape(NS, NCHUNK2, CH)
    colS = edge_index[1].reshape(NS, NCHUNK2, CH)
    zeros = jnp.zeros((RPW, FH), jnp.float32)

    wn3 = _wnorm_kernel(row3, col3, ew3)
    wnS = wn3.reshape(NS, NCHUNK2, CH)
    xh = jnp.stack([x[:, :FH], x[:, FH:]])
    P = _spmm_kernel(xh, rowS, colS, wnS, zeros)

    bn = 1000
    part = pl.pallas_call(
        _part_body,
        grid=(N // bn,),
        in_specs=[
            pl.BlockSpec((bn, F), lambda i: (i, 0)),
            pl.BlockSpec((NC, bn, FH), lambda i: (0, i, 0)),
            pl.BlockSpec((3, F, F), lambda i: (0, 0, 0)),
            pl.BlockSpec((1, F), lambda i: (0, 0)),
        ],
        out_specs=pl.BlockSpec((bn, F), lambda i: (i, 0)),
        out_shape=jax.ShapeDtypeStruct((N, F), jnp.float32),
    )(x, P, W, bias[None, :])

    U = _spmm_kernel(P, rowS, colS, wnS, zeros)

    out = pl.pallas_call(
        _fin_body,
        grid=(N // bn,),
        in_specs=[
            pl.BlockSpec((bn, F), lambda i: (i, 0)),
            pl.BlockSpec((NC, bn, FH), lambda i: (0, i, 0)),
            pl.BlockSpec((3, F, F), lambda i: (0, 0, 0)),
        ],
        out_specs=pl.BlockSpec((bn, F), lambda i: (i, 0)),
        out_shape=jax.ShapeDtypeStruct((N, F), jnp.float32),
    )(part, U, W)
    return out
